# baseline (speedup 1.0000x reference)
import functools

import jax
import jax.numpy as jnp
from jax import lax
from jax.experimental import pallas as pl
from jax.experimental.pallas import tpu as pltpu
from jax.experimental.pallas import tpu_sc as plsc

NC = 2
NS = 16
NW = NC * NS

CSUB = 32
SUB = 5
CHUNK = CSUB * SUB


def _table_body(w_ref, x_ref, o_ref):
    o_ref[...] = lax.dot_general(
        x_ref[...], w_ref[0],
        (((1,), (1,)), ((), ())),
        preferred_element_type=jnp.float32,
    )[None]


def _final_body(f_ref, c_ref, d_ref, wd_ref, wn_ref, b_ref, o_ref):
    s = f_ref[0] + f_ref[1]
    cnt = c_ref[0] + c_ref[1]
    hn = s / jnp.maximum(cnt, 1.0)
    o = lax.dot_general(d_ref[...], wd_ref[...], (((1,), (1,)), ((), ())),
                        preferred_element_type=jnp.float32)
    o += lax.dot_general(hn, wn_ref[...], (((1,), (1,)), ((), ())),
                         preferred_element_type=jnp.float32)
    o_ref[...] = jnp.maximum(o + b_ref[...], 0.0)


def _gidx_body(n_src, s_ref, r_ref, o_ref):
    o_ref[...] = r_ref[...] * n_src + s_ref[...]


def _sc_body(n_pad, e_per_w, tab, gidxi, dsti, zf,
             feat_out, cnt_out,
             gidx_v, dst_v, rows_v, ones_v, zbuf_v, sem,
             feat_acc, cnt_acc):
    cid = lax.axis_index("c")
    sid = lax.axis_index("s")
    wid = sid * NC + cid
    rows_per_tile = n_pad // NS
    nchunk = e_per_w // CHUNK

    for k in range(3):
        ones_v[pl.ds(k * 16, 16)] = jnp.ones((16,), jnp.float32)

    def zfill(k, _):
        zbuf_v[pl.ds(k * 16, 16)] = jnp.zeros((16,), jnp.float32)
        return _
    lax.fori_loop(0, zbuf_v.shape[0] // 16, zfill, None)

    r0 = sid * rows_per_tile
    pltpu.sync_copy(zf.at[pl.ds(r0, rows_per_tile)],
                    feat_acc.at[pl.ds(r0, rows_per_tile)])
    pltpu.sync_copy(zbuf_v.at[pl.ds(0, rows_per_tile)],
                    cnt_acc.at[pl.ds(r0, rows_per_tile)])
    plsc.subcore_barrier()

    def chunk(ch, _):
        e0 = wid * e_per_w + ch * CHUNK
        for j in range(SUB):
            pltpu.sync_copy(gidxi.at[pl.ds(e0 + j * CSUB, CSUB)], gidx_v.at[j])
            pltpu.sync_copy(dsti.at[pl.ds(e0 + j * CSUB, CSUB)], dst_v.at[j])

        descs = []
        for j in range(SUB):
            descs.append(pltpu.async_copy(
                tab.at[gidx_v.at[j]],
                rows_v.at[pl.ds(j * CSUB, CSUB)], sem))
        for d in descs:
            d.wait()
        for j in range(SUB):
            pltpu.sync_copy(rows_v.at[pl.ds(j * CSUB, CSUB)],
                            feat_acc.at[dst_v.at[j]], add=True)
            pltpu.sync_copy(ones_v.at[pl.ds(0, CSUB)],
                            cnt_acc.at[dst_v.at[j]], add=True)
        return _

    lax.fori_loop(0, nchunk, chunk, None)
    plsc.subcore_barrier()

    pltpu.sync_copy(feat_acc.at[pl.ds(r0, rows_per_tile)],
                    feat_out.at[cid, pl.ds(r0, rows_per_tile)])
    pltpu.sync_copy(cnt_acc.at[pl.ds(r0, rows_per_tile)],
                    cnt_out.at[pl.ds(cid * n_pad + r0, rows_per_tile)])


def kernel(src_features, dst_features, edge_index, rating, W_r, W_weight, W_bias):
    n_src, d = src_features.shape
    n_dst = dst_features.shape[0]
    e = rating.shape[0]
    r = W_r.shape[0]

    src_idx = edge_index[0].astype(jnp.int32)
    dst_idx = edge_index[1].astype(jnp.int32)
    rat = rating.astype(jnp.int32)

    bn = min(1000, n_src)
    table = pl.pallas_call(
        _table_body,
        grid=(r, n_src // bn),
        in_specs=[
            pl.BlockSpec((1, d, d), lambda ri, bi: (ri, 0, 0)),
            pl.BlockSpec((bn, d), lambda ri, bi: (bi, 0)),
        ],
        out_specs=pl.BlockSpec((1, bn, d), lambda ri, bi: (ri, bi, 0)),
        out_shape=jax.ShapeDtypeStruct((r, n_src, d), jnp.float32),
    )(W_r, src_features)
    table = table.reshape(r * n_src, d)

    ecols = 128
    erows = e // ecols
    gidx = pl.pallas_call(
        functools.partial(_gidx_body, n_src),
        grid=(1,),
        in_specs=[
            pl.BlockSpec((erows, ecols), lambda bi: (0, 0)),
            pl.BlockSpec((erows, ecols), lambda bi: (0, 0)),
        ],
        out_specs=pl.BlockSpec((erows, ecols), lambda bi: (0, 0)),
        out_shape=jax.ShapeDtypeStruct((erows, ecols), jnp.int32),
    )(src_idx.reshape(erows, ecols), rat.reshape(erows, ecols))
    gidx = gidx.reshape(e)

    e_per_w = e // NW
    n_pad = (n_dst // 1024 + 1) * 1024
    epw_pad = ((e_per_w + CHUNK - 1) // CHUNK) * CHUNK
    g1d = jnp.pad(gidx.reshape(NW, e_per_w),
                  ((0, 0), (0, epw_pad - e_per_w))).reshape(-1)
    d1d = jnp.pad(dst_idx.reshape(NW, e_per_w),
                  ((0, 0), (0, epw_pad - e_per_w)),
                  constant_values=n_dst).reshape(-1)
    zf = jnp.zeros((n_pad, d), jnp.float32)

    mesh = plsc.VectorSubcoreMesh(core_axis_name="c", subcore_axis_name="s")
    sc_fn = pl.kernel(
        functools.partial(_sc_body, n_pad, epw_pad),
        out_type=(
            jax.ShapeDtypeStruct((NC, n_pad, d), jnp.float32),
            jax.ShapeDtypeStruct((NC * n_pad,), jnp.float32),
        ),
        mesh=mesh,
        scratch_types=[
            pltpu.VMEM((SUB, CSUB), jnp.int32),
            pltpu.VMEM((SUB, CSUB), jnp.int32),
            pltpu.VMEM((CHUNK, d), jnp.float32),
            pltpu.VMEM((48,), jnp.float32),
            pltpu.VMEM((640,), jnp.float32),
            pltpu.SemaphoreType.DMA,
            pltpu.VMEM_SHARED((n_pad, d), jnp.float32),
            pltpu.VMEM_SHARED((n_pad,), jnp.float32),
        ],
    )
    feat_p, cnt_p = sc_fn(table, g1d, d1d, zf)
    cnt_p = cnt_p.reshape(NC, n_pad, 1)

    wd = W_weight[:, :d]
    wn = W_weight[:, d:]
    bias = W_bias.reshape(1, d)
    bf = 1024
    dst_pad = jnp.pad(dst_features, ((0, n_pad - n_dst), (0, 0)))
    out = pl.pallas_call(
        _final_body,
        grid=(n_pad // bf,),
        in_specs=[
            pl.BlockSpec((NC, bf, d), lambda bi: (0, bi, 0)),
            pl.BlockSpec((NC, bf, 1), lambda bi: (0, bi, 0)),
            pl.BlockSpec((bf, d), lambda bi: (bi, 0)),
            pl.BlockSpec((d, d), lambda bi: (0, 0)),
            pl.BlockSpec((d, d), lambda bi: (0, 0)),
            pl.BlockSpec((1, d), lambda bi: (0, 0)),
        ],
        out_specs=pl.BlockSpec((bf, d), lambda bi: (bi, 0)),
        out_shape=jax.ShapeDtypeStruct((n_pad, d), jnp.float32),
    )(feat_p, cnt_p, dst_pad, wd, wn, bias)
    return out[:n_dst]


# speedup vs baseline: 1.9920x; 1.9920x over previous
import functools

import jax
import jax.numpy as jnp
from jax import lax
from jax.experimental import pallas as pl
from jax.experimental.pallas import tpu as pltpu
from jax.experimental.pallas import tpu_sc as plsc

NC = 2
NS = 16
NW = NC * NS

CSUB = 40
SUB = 5
CHUNK = CSUB * SUB


def _table_body(w_ref, x_ref, o_ref):
    o_ref[...] = lax.dot_general(
        x_ref[...], w_ref[0],
        (((1,), (1,)), ((), ())),
        preferred_element_type=jnp.float32,
    )[None]


def _final_body(f_ref, c_ref, d_ref, wd_ref, wn_ref, b_ref, o_ref):
    s = f_ref[0] + f_ref[1]
    cnt = c_ref[0] + c_ref[1]
    hn = s / jnp.maximum(cnt, 1.0)
    o = lax.dot_general(d_ref[...], wd_ref[...], (((1,), (1,)), ((), ())),
                        preferred_element_type=jnp.float32)
    o += lax.dot_general(hn, wn_ref[...], (((1,), (1,)), ((), ())),
                         preferred_element_type=jnp.float32)
    o_ref[...] = jnp.maximum(o + b_ref[...], 0.0)


def _gidx_body(n_src, s_ref, r_ref, o_ref):
    o_ref[...] = r_ref[...] * n_src + s_ref[...]


def _sc_body(n_pad, e_per_w, tab, gidxi, dsti, zf,
             feat_out, cnt_out,
             gidx_v, dst_v, rows_v, ones_v, zbuf_v, sem, sem_i,
             feat_acc, cnt_acc):
    cid = lax.axis_index("c")
    sid = lax.axis_index("s")
    wid = sid * NC + cid
    rows_per_tile = n_pad // NS
    nchunk = e_per_w // CHUNK

    for k in range(3):
        ones_v[pl.ds(k * 16, 16)] = jnp.ones((16,), jnp.float32)

    def zfill(k, _):
        zbuf_v[pl.ds(k * 16, 16)] = jnp.zeros((16,), jnp.float32)
        return _
    lax.fori_loop(0, zbuf_v.shape[0] // 16, zfill, None)

    r0 = sid * rows_per_tile
    pltpu.sync_copy(zf.at[pl.ds(r0, rows_per_tile)],
                    feat_acc.at[pl.ds(r0, rows_per_tile)])
    pltpu.sync_copy(zbuf_v.at[pl.ds(0, rows_per_tile)],
                    cnt_acc.at[pl.ds(r0, rows_per_tile)])
    plsc.subcore_barrier()

    def chunk(ch, _):
        e0 = wid * e_per_w + ch * CHUNK
        idescs = []
        for j in range(SUB):
            idescs.append(pltpu.async_copy(
                gidxi.at[pl.ds(e0 + j * CSUB, CSUB)], gidx_v.at[j], sem_i))
            idescs.append(pltpu.async_copy(
                dsti.at[pl.ds(e0 + j * CSUB, CSUB)], dst_v.at[j], sem_i))
        for d in idescs:
            d.wait()

        descs = []
        for j in range(SUB):
            descs.append(pltpu.async_copy(
                tab.at[gidx_v.at[j]],
                rows_v.at[pl.ds(j * CSUB, CSUB)], sem))
        for d in descs:
            d.wait()
        for j in range(SUB):
            pltpu.sync_copy(rows_v.at[pl.ds(j * CSUB, CSUB)],
                            feat_acc.at[dst_v.at[j]], add=True)
            pltpu.sync_copy(ones_v.at[pl.ds(0, CSUB)],
                            cnt_acc.at[dst_v.at[j]], add=True)
        return _

    lax.fori_loop(0, nchunk, chunk, None)
    plsc.subcore_barrier()

    pltpu.sync_copy(feat_acc.at[pl.ds(r0, rows_per_tile)],
                    feat_out.at[cid, pl.ds(r0, rows_per_tile)])
    pltpu.sync_copy(cnt_acc.at[pl.ds(r0, rows_per_tile)],
                    cnt_out.at[pl.ds(cid * n_pad + r0, rows_per_tile)])


def kernel(src_features, dst_features, edge_index, rating, W_r, W_weight, W_bias):
    n_src, d = src_features.shape
    n_dst = dst_features.shape[0]
    e = rating.shape[0]
    r = W_r.shape[0]

    src_idx = edge_index[0].astype(jnp.int32)
    dst_idx = edge_index[1].astype(jnp.int32)
    rat = rating.astype(jnp.int32)

    bn = min(1000, n_src)
    table = pl.pallas_call(
        _table_body,
        grid=(r, n_src // bn),
        in_specs=[
            pl.BlockSpec((1, d, d), lambda ri, bi: (ri, 0, 0)),
            pl.BlockSpec((bn, d), lambda ri, bi: (bi, 0)),
        ],
        out_specs=pl.BlockSpec((1, bn, d), lambda ri, bi: (ri, bi, 0)),
        out_shape=jax.ShapeDtypeStruct((r, n_src, d), jnp.float32),
    )(W_r, src_features)
    table = table.reshape(r * n_src, d)

    ecols = 128
    erows = e // ecols
    gidx = pl.pallas_call(
        functools.partial(_gidx_body, n_src),
        grid=(1,),
        in_specs=[
            pl.BlockSpec((erows, ecols), lambda bi: (0, 0)),
            pl.BlockSpec((erows, ecols), lambda bi: (0, 0)),
        ],
        out_specs=pl.BlockSpec((erows, ecols), lambda bi: (0, 0)),
        out_shape=jax.ShapeDtypeStruct((erows, ecols), jnp.int32),
    )(src_idx.reshape(erows, ecols), rat.reshape(erows, ecols))
    gidx = gidx.reshape(e)

    e_per_w = e // NW
    n_pad = ((n_dst + 1023) // 1024) * 1024
    zf = jnp.zeros((n_pad, d), jnp.float32)

    mesh = plsc.VectorSubcoreMesh(core_axis_name="c", subcore_axis_name="s")
    sc_fn = pl.kernel(
        functools.partial(_sc_body, n_pad, e_per_w),
        out_type=(
            jax.ShapeDtypeStruct((NC, n_pad, d), jnp.float32),
            jax.ShapeDtypeStruct((NC * n_pad,), jnp.float32),
        ),
        mesh=mesh,
        scratch_types=[
            pltpu.VMEM((SUB, CSUB), jnp.int32),
            pltpu.VMEM((SUB, CSUB), jnp.int32),
            pltpu.VMEM((CHUNK, d), jnp.float32),
            pltpu.VMEM((48,), jnp.float32),
            pltpu.VMEM((640,), jnp.float32),
            pltpu.SemaphoreType.DMA,
            pltpu.SemaphoreType.DMA,
            pltpu.VMEM_SHARED((n_pad, d), jnp.float32),
            pltpu.VMEM_SHARED((n_pad,), jnp.float32),
        ],
    )
    feat_p, cnt_p = sc_fn(table, gidx, dst_idx, zf)
    cnt_p = cnt_p.reshape(NC, n_pad, 1)

    wd = W_weight[:, :d]
    wn = W_weight[:, d:]
    bias = W_bias.reshape(1, d)
    bf = 1024
    dst_pad = jnp.pad(dst_features, ((0, n_pad - n_dst), (0, 0)))
    out = pl.pallas_call(
        _final_body,
        grid=(n_pad // bf,),
        in_specs=[
            pl.BlockSpec((NC, bf, d), lambda bi: (0, bi, 0)),
            pl.BlockSpec((NC, bf, 1), lambda bi: (0, bi, 0)),
            pl.BlockSpec((bf, d), lambda bi: (bi, 0)),
            pl.BlockSpec((d, d), lambda bi: (0, 0)),
            pl.BlockSpec((d, d), lambda bi: (0, 0)),
            pl.BlockSpec((1, d), lambda bi: (0, 0)),
        ],
        out_specs=pl.BlockSpec((bf, d), lambda bi: (bi, 0)),
        out_shape=jax.ShapeDtypeStruct((n_pad, d), jnp.float32),
    )(feat_p, cnt_p, dst_pad, wd, wn, bias)
    return out[:n_dst]


# speedup vs baseline: 2.1756x; 1.0922x over previous
import functools

import jax
import jax.numpy as jnp
from jax import lax
from jax.experimental import pallas as pl
from jax.experimental.pallas import tpu as pltpu
from jax.experimental.pallas import tpu_sc as plsc

NC = 2
NS = 16
NW = NC * NS

CSUB = 40
SUB = 5
CHUNK = CSUB * SUB


def _table_body(w_ref, x_ref, o_ref):
    o_ref[...] = lax.dot_general(
        x_ref[...], w_ref[0],
        (((1,), (1,)), ((), ())),
        preferred_element_type=jnp.float32,
    )[None]


def _final_body(f_ref, c_ref, d_ref, wd_ref, wn_ref, b_ref, o_ref):
    s = f_ref[0] + f_ref[1]
    cnt = c_ref[0] + c_ref[1]
    hn = s / jnp.maximum(cnt, 1.0)
    o = lax.dot_general(d_ref[...], wd_ref[...], (((1,), (1,)), ((), ())),
                        preferred_element_type=jnp.float32)
    o += lax.dot_general(hn, wn_ref[...], (((1,), (1,)), ((), ())),
                         preferred_element_type=jnp.float32)
    o_ref[...] = jnp.maximum(o + b_ref[...], 0.0)


def _gidx_body(n_src, s_ref, r_ref, o_ref):
    o_ref[...] = r_ref[...] * n_src + s_ref[...]


def _sc_body(n_pad, e_per_w, tab, gidxi, dsti, zf,
             feat_out, cnt_out,
             gidx_v, dst_v, rows_v, ones_v, zbuf_v, sem, sem_i, sem_s,
             feat_acc, cnt_acc):
    cid = lax.axis_index("c")
    sid = lax.axis_index("s")
    wid = sid * NC + cid
    rows_per_tile = n_pad // NS
    nchunk = e_per_w // CHUNK

    for k in range(3):
        ones_v[pl.ds(k * 16, 16)] = jnp.ones((16,), jnp.float32)

    def zfill(k, _):
        zbuf_v[pl.ds(k * 16, 16)] = jnp.zeros((16,), jnp.float32)
        return _
    lax.fori_loop(0, zbuf_v.shape[0] // 16, zfill, None)

    r0 = sid * rows_per_tile
    pltpu.sync_copy(zf.at[pl.ds(r0, rows_per_tile)],
                    feat_acc.at[pl.ds(r0, rows_per_tile)])
    pltpu.sync_copy(zbuf_v.at[pl.ds(0, rows_per_tile)],
                    cnt_acc.at[pl.ds(r0, rows_per_tile)])
    plsc.subcore_barrier()

    def chunk(ch, _):
        e0 = wid * e_per_w + ch * CHUNK
        idescs = []
        for j in range(SUB):
            idescs.append(pltpu.async_copy(
                gidxi.at[pl.ds(e0 + j * CSUB, CSUB)], gidx_v.at[j], sem_i))
            idescs.append(pltpu.async_copy(
                dsti.at[pl.ds(e0 + j * CSUB, CSUB)], dst_v.at[j], sem_i))
        for d in idescs:
            d.wait()

        descs = []
        for j in range(SUB):
            descs.append(pltpu.async_copy(
                tab.at[gidx_v.at[j]],
                rows_v.at[pl.ds(j * CSUB, CSUB)], sem))
        for d in descs:
            d.wait()
        sdescs = []
        for j in range(SUB):
            sdescs.append(pltpu.async_copy(
                rows_v.at[pl.ds(j * CSUB, CSUB)],
                feat_acc.at[dst_v.at[j]], sem_s, add=True))
            sdescs.append(pltpu.async_copy(
                ones_v.at[pl.ds(0, CSUB)],
                cnt_acc.at[dst_v.at[j]], sem_s, add=True))
        for d in sdescs:
            d.wait()
        return _

    lax.fori_loop(0, nchunk, chunk, None)
    plsc.subcore_barrier()

    pltpu.sync_copy(feat_acc.at[pl.ds(r0, rows_per_tile)],
                    feat_out.at[cid, pl.ds(r0, rows_per_tile)])
    pltpu.sync_copy(cnt_acc.at[pl.ds(r0, rows_per_tile)],
                    cnt_out.at[pl.ds(cid * n_pad + r0, rows_per_tile)])


def kernel(src_features, dst_features, edge_index, rating, W_r, W_weight, W_bias):
    n_src, d = src_features.shape
    n_dst = dst_features.shape[0]
    e = rating.shape[0]
    r = W_r.shape[0]

    src_idx = edge_index[0].astype(jnp.int32)
    dst_idx = edge_index[1].astype(jnp.int32)
    rat = rating.astype(jnp.int32)

    bn = min(1000, n_src)
    table = pl.pallas_call(
        _table_body,
        grid=(r, n_src // bn),
        in_specs=[
            pl.BlockSpec((1, d, d), lambda ri, bi: (ri, 0, 0)),
            pl.BlockSpec((bn, d), lambda ri, bi: (bi, 0)),
        ],
        out_specs=pl.BlockSpec((1, bn, d), lambda ri, bi: (ri, bi, 0)),
        out_shape=jax.ShapeDtypeStruct((r, n_src, d), jnp.float32),
    )(W_r, src_features)
    table = table.reshape(r * n_src, d)

    ecols = 128
    erows = e // ecols
    gidx = pl.pallas_call(
        functools.partial(_gidx_body, n_src),
        grid=(1,),
        in_specs=[
            pl.BlockSpec((erows, ecols), lambda bi: (0, 0)),
            pl.BlockSpec((erows, ecols), lambda bi: (0, 0)),
        ],
        out_specs=pl.BlockSpec((erows, ecols), lambda bi: (0, 0)),
        out_shape=jax.ShapeDtypeStruct((erows, ecols), jnp.int32),
    )(src_idx.reshape(erows, ecols), rat.reshape(erows, ecols))
    gidx = gidx.reshape(e)

    e_per_w = e // NW
    n_pad = ((n_dst + 1023) // 1024) * 1024
    zf = jnp.zeros((n_pad, d), jnp.float32)

    mesh = plsc.VectorSubcoreMesh(core_axis_name="c", subcore_axis_name="s")
    sc_fn = pl.kernel(
        functools.partial(_sc_body, n_pad, e_per_w),
        out_type=(
            jax.ShapeDtypeStruct((NC, n_pad, d), jnp.float32),
            jax.ShapeDtypeStruct((NC * n_pad,), jnp.float32),
        ),
        mesh=mesh,
        scratch_types=[
            pltpu.VMEM((SUB, CSUB), jnp.int32),
            pltpu.VMEM((SUB, CSUB), jnp.int32),
            pltpu.VMEM((CHUNK, d), jnp.float32),
            pltpu.VMEM((48,), jnp.float32),
            pltpu.VMEM((640,), jnp.float32),
            pltpu.SemaphoreType.DMA,
            pltpu.SemaphoreType.DMA,
            pltpu.SemaphoreType.DMA,
            pltpu.VMEM_SHARED((n_pad, d), jnp.float32),
            pltpu.VMEM_SHARED((n_pad,), jnp.float32),
        ],
    )
    feat_p, cnt_p = sc_fn(table, gidx, dst_idx, zf)
    cnt_p = cnt_p.reshape(NC, n_pad, 1)

    wd = W_weight[:, :d]
    wn = W_weight[:, d:]
    bias = W_bias.reshape(1, d)
    bf = 1024
    dst_pad = jnp.pad(dst_features, ((0, n_pad - n_dst), (0, 0)))
    out = pl.pallas_call(
        _final_body,
        grid=(n_pad // bf,),
        in_specs=[
            pl.BlockSpec((NC, bf, d), lambda bi: (0, bi, 0)),
            pl.BlockSpec((NC, bf, 1), lambda bi: (0, bi, 0)),
            pl.BlockSpec((bf, d), lambda bi: (bi, 0)),
            pl.BlockSpec((d, d), lambda bi: (0, 0)),
            pl.BlockSpec((d, d), lambda bi: (0, 0)),
            pl.BlockSpec((1, d), lambda bi: (0, 0)),
        ],
        out_specs=pl.BlockSpec((bf, d), lambda bi: (bi, 0)),
        out_shape=jax.ShapeDtypeStruct((n_pad, d), jnp.float32),
    )(feat_p, cnt_p, dst_pad, wd, wn, bias)
    return out[:n_dst]


# speedup vs baseline: 2.3003x; 1.0573x over previous
import functools

import jax
import jax.numpy as jnp
from jax import lax
from jax.experimental import pallas as pl
from jax.experimental.pallas import tpu as pltpu
from jax.experimental.pallas import tpu_sc as plsc

NC = 2
NS = 16
NW = NC * NS

CSUB = 40
SUB = 5
CHUNK = CSUB * SUB


def _table_body(w_ref, x_ref, o_ref):
    o_ref[...] = lax.dot_general(
        x_ref[...], w_ref[0],
        (((1,), (1,)), ((), ())),
        preferred_element_type=jnp.float32,
    )[None]


def _final_body(f_ref, c_ref, d_ref, wd_ref, wn_ref, b_ref, o_ref):
    s = f_ref[0] + f_ref[1]
    cnt = c_ref[0] + c_ref[1]
    hn = s / jnp.maximum(cnt, 1.0)
    o = lax.dot_general(d_ref[...], wd_ref[...], (((1,), (1,)), ((), ())),
                        preferred_element_type=jnp.float32)
    o += lax.dot_general(hn, wn_ref[...], (((1,), (1,)), ((), ())),
                         preferred_element_type=jnp.float32)
    o_ref[...] = jnp.maximum(o + b_ref[...], 0.0)


def _gidx_body(n_src, s_ref, r_ref, o_ref):
    o_ref[...] = r_ref[...] * n_src + s_ref[...]


def _sc_body(n_pad, e_per_w, tab, gidxi, dsti, zf,
             feat_out, cnt_out,
             gidx_v, dst_v, rows_v, ones_v, zbuf_v, sem, sem_i, sem_s,
             feat_acc, cnt_acc):
    cid = lax.axis_index("c")
    sid = lax.axis_index("s")
    wid = sid * NC + cid
    rows_per_tile = n_pad // NS
    nchunk = e_per_w // CHUNK

    for k in range(3):
        ones_v[pl.ds(k * 16, 16)] = jnp.ones((16,), jnp.float32)

    def zfill(k, _):
        zbuf_v[pl.ds(k * 16, 16)] = jnp.zeros((16,), jnp.float32)
        return _
    lax.fori_loop(0, zbuf_v.shape[0] // 16, zfill, None)

    r0 = sid * rows_per_tile
    pltpu.sync_copy(zf.at[pl.ds(r0, rows_per_tile)],
                    feat_acc.at[pl.ds(r0, rows_per_tile)])
    pltpu.sync_copy(zbuf_v.at[pl.ds(0, rows_per_tile)],
                    cnt_acc.at[pl.ds(r0, rows_per_tile)])
    plsc.subcore_barrier()

    def chunk(ch, _):
        e0 = wid * e_per_w + ch * CHUNK
        idescs = []
        for j in range(SUB):
            idescs.append(pltpu.async_copy(
                gidxi.at[pl.ds(e0 + j * CSUB, CSUB)], gidx_v.at[j], sem_i))
            idescs.append(pltpu.async_copy(
                dsti.at[pl.ds(e0 + j * CSUB, CSUB)], dst_v.at[j], sem_i))
        for d in idescs:
            d.wait()

        descs = []
        for j in range(SUB):
            descs.append(pltpu.async_copy(
                tab.at[gidx_v.at[j]],
                rows_v.at[pl.ds(j * CSUB, CSUB)], sem))
        for d in descs:
            d.wait()
        sdescs = []
        for j in range(SUB):
            sdescs.append(pltpu.async_copy(
                rows_v.at[pl.ds(j * CSUB, CSUB)],
                feat_acc.at[dst_v.at[j]], sem_s, add=True))
            sdescs.append(pltpu.async_copy(
                ones_v.at[pl.ds(0, CSUB)],
                cnt_acc.at[dst_v.at[j]], sem_s, add=True))
        for d in sdescs:
            d.wait()
        return _

    lax.fori_loop(0, nchunk, chunk, None)
    plsc.subcore_barrier()

    pltpu.sync_copy(feat_acc.at[pl.ds(r0, rows_per_tile)],
                    feat_out.at[cid, pl.ds(r0, rows_per_tile)])
    pltpu.sync_copy(cnt_acc.at[pl.ds(r0, rows_per_tile)],
                    cnt_out.at[pl.ds(cid * n_pad + r0, rows_per_tile)])


def kernel(src_features, dst_features, edge_index, rating, W_r, W_weight, W_bias):
    n_src, d = src_features.shape
    n_dst = dst_features.shape[0]
    e = rating.shape[0]
    r = W_r.shape[0]

    src_idx = edge_index[0].astype(jnp.int32)
    dst_idx = edge_index[1].astype(jnp.int32)
    rat = rating.astype(jnp.int32)

    bn = min(2000, n_src)
    table = pl.pallas_call(
        _table_body,
        grid=(r, n_src // bn),
        in_specs=[
            pl.BlockSpec((1, d, d), lambda ri, bi: (ri, 0, 0)),
            pl.BlockSpec((bn, d), lambda ri, bi: (bi, 0)),
        ],
        out_specs=pl.BlockSpec((1, bn, d), lambda ri, bi: (ri, bi, 0)),
        out_shape=jax.ShapeDtypeStruct((r, n_src, d), jnp.float32),
    )(W_r, src_features)
    table = table.reshape(r * n_src, d)

    ecols = 128
    erows = e // ecols
    gidx = pl.pallas_call(
        functools.partial(_gidx_body, n_src),
        grid=(1,),
        in_specs=[
            pl.BlockSpec((erows, ecols), lambda bi: (0, 0)),
            pl.BlockSpec((erows, ecols), lambda bi: (0, 0)),
        ],
        out_specs=pl.BlockSpec((erows, ecols), lambda bi: (0, 0)),
        out_shape=jax.ShapeDtypeStruct((erows, ecols), jnp.int32),
    )(src_idx.reshape(erows, ecols), rat.reshape(erows, ecols))
    gidx = gidx.reshape(e)

    e_per_w = e // NW
    n_pad = ((n_dst + 1023) // 1024) * 1024
    zf = jnp.zeros((n_pad, d), jnp.float32)

    mesh = plsc.VectorSubcoreMesh(core_axis_name="c", subcore_axis_name="s")
    sc_fn = pl.kernel(
        functools.partial(_sc_body, n_pad, e_per_w),
        out_type=(
            jax.ShapeDtypeStruct((NC, n_pad, d), jnp.float32),
            jax.ShapeDtypeStruct((NC * n_pad,), jnp.float32),
        ),
        mesh=mesh,
        scratch_types=[
            pltpu.VMEM((SUB, CSUB), jnp.int32),
            pltpu.VMEM((SUB, CSUB), jnp.int32),
            pltpu.VMEM((CHUNK, d), jnp.float32),
            pltpu.VMEM((48,), jnp.float32),
            pltpu.VMEM((640,), jnp.float32),
            pltpu.SemaphoreType.DMA,
            pltpu.SemaphoreType.DMA,
            pltpu.SemaphoreType.DMA,
            pltpu.VMEM_SHARED((n_pad, d), jnp.float32),
            pltpu.VMEM_SHARED((n_pad,), jnp.float32),
        ],
    )
    feat_p, cnt_p = sc_fn(table, gidx, dst_idx, zf)
    cnt_p = cnt_p.reshape(NC, n_pad, 1)

    wd = W_weight[:, :d]
    wn = W_weight[:, d:]
    bias = W_bias.reshape(1, d)
    bf = 1024
    dst_pad = jnp.pad(dst_features, ((0, n_pad - n_dst), (0, 0)))
    out = pl.pallas_call(
        _final_body,
        grid=(n_pad // bf,),
        in_specs=[
            pl.BlockSpec((NC, bf, d), lambda bi: (0, bi, 0)),
            pl.BlockSpec((NC, bf, 1), lambda bi: (0, bi, 0)),
            pl.BlockSpec((bf, d), lambda bi: (bi, 0)),
            pl.BlockSpec((d, d), lambda bi: (0, 0)),
            pl.BlockSpec((d, d), lambda bi: (0, 0)),
            pl.BlockSpec((1, d), lambda bi: (0, 0)),
        ],
        out_specs=pl.BlockSpec((bf, d), lambda bi: (bi, 0)),
        out_shape=jax.ShapeDtypeStruct((n_pad, d), jnp.float32),
    )(feat_p, cnt_p, dst_pad, wd, wn, bias)
    return out[:n_dst]
